# baseline (speedup 1.0000x reference)
import functools

import jax
import jax.numpy as jnp
from jax import lax
from jax.experimental import pallas as pl
from jax.experimental.pallas import tpu as pltpu
from jax.experimental.pallas import tpu_sc as plsc

GRID = 16
N_CENTERS = 256
DIM = 512
BATCH = 4096

NC = 2
NS = 16
L = 16
NW = NC * NS
B_PER_W = BATCH // NW
CHUNK = 64
N_CHUNKS = B_PER_W // CHUNK
N_ACC = 8


def _make_sc_kernel():
    mesh = plsc.VectorSubcoreMesh(core_axis_name="c", subcore_axis_name="s")

    @functools.partial(
        pl.kernel,
        mesh=mesh,
        out_type=jax.ShapeDtypeStruct((NW, L), jnp.float32),
        scratch_types=[
            pltpu.VMEM((CHUNK,), jnp.int32),
            pltpu.VMEM((CHUNK,), jnp.int32),
            pltpu.VMEM((CHUNK,), jnp.int32),
            pltpu.VMEM((CHUNK, DIM), jnp.float32),
            pltpu.VMEM((CHUNK, DIM), jnp.float32),
            pltpu.VMEM((L,), jnp.float32),
            pltpu.SemaphoreType.DMA,
            pltpu.SemaphoreType.DMA,
        ],
    )
    def sc_loss(x_hbm, coords_hbm, centers_hbm, out_hbm,
                lon_v, lat_v, idx_v, x_buf, c_buf, acc_v, sem_g, sem_x):
        wid = lax.axis_index("s") * NC + lax.axis_index("c")
        accs = tuple(jnp.zeros((L,), jnp.float32) for _ in range(N_ACC))

        for ch in range(N_CHUNKS):
            base = wid * B_PER_W + ch * CHUNK
            pltpu.sync_copy(coords_hbm.at[pl.ds(base, CHUNK)], lon_v)
            pltpu.sync_copy(coords_hbm.at[pl.ds(BATCH + base, CHUNK)], lat_v)
            for j in range(CHUNK // L):
                s = pl.ds(j * L, L)
                idx_v[s] = lon_v[s] * GRID + lat_v[s]
            cp_g = pltpu.async_copy(centers_hbm.at[idx_v], c_buf, sem_g)
            cp_x = pltpu.async_copy(x_hbm.at[pl.ds(base, CHUNK)], x_buf, sem_x)
            cp_g.wait()
            cp_x.wait()

            def body(r, acc):
                acc = list(acc)
                for j in range(DIM // L):
                    s = pl.ds(j * L, L)
                    d = x_buf[r, s] - c_buf[r, s]
                    acc[j % N_ACC] = acc[j % N_ACC] + d * d
                return tuple(acc)

            accs = lax.fori_loop(0, CHUNK, body, accs)

        total = accs[0]
        for a in accs[1:]:
            total = total + a
        acc_v[...] = total
        pltpu.sync_copy(acc_v, out_hbm.at[wid])

    return sc_loss


_sc_loss = _make_sc_kernel()


def kernel(batch_tensors, batch_coords, cluster_centers):
    coords_flat = batch_coords.reshape(-1)
    partials = _sc_loss(batch_tensors, coords_flat, cluster_centers)
    return jnp.sum(partials) / jnp.float32(BATCH * DIM)


# speedup vs baseline: 1.0799x; 1.0799x over previous
import functools

import jax
import jax.numpy as jnp
from jax import lax
from jax.experimental import pallas as pl
from jax.experimental.pallas import tpu as pltpu
from jax.experimental.pallas import tpu_sc as plsc

GRID = 16
N_CENTERS = 256
DIM = 512
BATCH = 4096

NC = 2
NS = 16
L = 16
NW = NC * NS
B_PER_W = BATCH // NW
CHUNK = 32
N_CHUNKS = B_PER_W // CHUNK
N_ACC = 8


def _make_sc_kernel():
    mesh = plsc.VectorSubcoreMesh(core_axis_name="c", subcore_axis_name="s")

    @functools.partial(
        pl.kernel,
        mesh=mesh,
        out_type=jax.ShapeDtypeStruct((NW, L), jnp.float32),
        scratch_types=[
            pltpu.VMEM((CHUNK,), jnp.int32),
            pltpu.VMEM((CHUNK,), jnp.int32),
            pltpu.VMEM((2, CHUNK), jnp.int32),
            pltpu.VMEM((2, CHUNK, DIM), jnp.float32),
            pltpu.VMEM((2, CHUNK, DIM), jnp.float32),
            pltpu.VMEM((L,), jnp.float32),
            pltpu.SemaphoreType.DMA,
            pltpu.SemaphoreType.DMA,
            pltpu.SemaphoreType.DMA,
            pltpu.SemaphoreType.DMA,
        ],
    )
    def sc_loss(x_hbm, coords_hbm, centers_hbm, out_hbm,
                lon_v, lat_v, idx_v, x_buf, c_buf, acc_v,
                sem_g0, sem_g1, sem_x0, sem_x1):
        wid = lax.axis_index("s") * NC + lax.axis_index("c")
        sem_g = (sem_g0, sem_g1)
        sem_x = (sem_x0, sem_x1)

        def issue(ch):
            slot = ch % 2
            base = wid * B_PER_W + ch * CHUNK
            pltpu.sync_copy(coords_hbm.at[pl.ds(base, CHUNK)], lon_v)
            pltpu.sync_copy(coords_hbm.at[pl.ds(BATCH + base, CHUNK)], lat_v)
            for j in range(CHUNK // L):
                s = pl.ds(j * L, L)
                idx_v[slot, s] = lon_v[s] * GRID + lat_v[s]
            cp_g = pltpu.async_copy(centers_hbm.at[idx_v.at[slot]],
                                    c_buf.at[slot], sem_g[slot])
            cp_x = pltpu.async_copy(x_hbm.at[pl.ds(base, CHUNK)],
                                    x_buf.at[slot], sem_x[slot])
            return cp_g, cp_x

        accs = tuple(jnp.zeros((L,), jnp.float32) for _ in range(N_ACC))
        pending = issue(0)
        for ch in range(N_CHUNKS):
            slot = ch % 2
            cp_g, cp_x = pending
            if ch + 1 < N_CHUNKS:
                pending = issue(ch + 1)
            cp_g.wait()
            cp_x.wait()

            xb = x_buf.at[slot]
            cb = c_buf.at[slot]

            def body(r, acc, xb=xb, cb=cb):
                acc = list(acc)
                for j in range(DIM // L):
                    s = pl.ds(j * L, L)
                    d = xb[r, s] - cb[r, s]
                    acc[j % N_ACC] = acc[j % N_ACC] + d * d
                return tuple(acc)

            accs = lax.fori_loop(0, CHUNK, body, accs)

        total = accs[0]
        for a in accs[1:]:
            total = total + a
        acc_v[...] = total
        pltpu.sync_copy(acc_v, out_hbm.at[wid])

    return sc_loss


_sc_loss = _make_sc_kernel()


def kernel(batch_tensors, batch_coords, cluster_centers):
    coords_flat = batch_coords.reshape(-1)
    partials = _sc_loss(batch_tensors, coords_flat, cluster_centers)
    return jnp.sum(partials) / jnp.float32(BATCH * DIM)


# speedup vs baseline: 1.3432x; 1.2438x over previous
import functools

import jax
import jax.numpy as jnp
from jax import lax
from jax.experimental import pallas as pl
from jax.experimental.pallas import tpu as pltpu
from jax.experimental.pallas import tpu_sc as plsc

GRID = 16
N_CENTERS = 256
DIM = 512
BATCH = 4096

TC_SHARE = 3072
SC_SHARE = BATCH - TC_SHARE
TC_BLOCK = 512

NC = 2
NS = 16
L = 16
NW = NC * NS
B_PER_W = SC_SHARE // NW
N_ACC = 8


def _make_sc_kernel():
    mesh = plsc.VectorSubcoreMesh(core_axis_name="c", subcore_axis_name="s")

    @functools.partial(
        pl.kernel,
        mesh=mesh,
        out_type=jax.ShapeDtypeStruct((NW, L), jnp.float32),
        scratch_types=[
            pltpu.VMEM((B_PER_W,), jnp.int32),
            pltpu.VMEM((B_PER_W,), jnp.int32),
            pltpu.VMEM((B_PER_W,), jnp.int32),
            pltpu.VMEM((B_PER_W, DIM), jnp.float32),
            pltpu.VMEM((B_PER_W, DIM), jnp.float32),
            pltpu.VMEM((L,), jnp.float32),
            pltpu.SemaphoreType.DMA,
            pltpu.SemaphoreType.DMA,
        ],
    )
    def sc_loss(x_hbm, coords_hbm, centers_hbm, out_hbm,
                lon_v, lat_v, idx_v, x_buf, c_buf, acc_v, sem_g, sem_x):
        wid = lax.axis_index("s") * NC + lax.axis_index("c")
        base = TC_SHARE + wid * B_PER_W
        pltpu.sync_copy(coords_hbm.at[pl.ds(base, B_PER_W)], lon_v)
        pltpu.sync_copy(coords_hbm.at[pl.ds(BATCH + base, B_PER_W)], lat_v)
        for j in range(B_PER_W // L):
            s = pl.ds(j * L, L)
            idx_v[s] = lon_v[s] * GRID + lat_v[s]
        cp_g = pltpu.async_copy(centers_hbm.at[idx_v], c_buf, sem_g)
        cp_x = pltpu.async_copy(x_hbm.at[pl.ds(base, B_PER_W)], x_buf, sem_x)
        cp_g.wait()
        cp_x.wait()

        def body(r, acc):
            acc = list(acc)
            for j in range(DIM // L):
                s = pl.ds(j * L, L)
                d = x_buf[r, s] - c_buf[r, s]
                acc[j % N_ACC] = acc[j % N_ACC] + d * d
            return tuple(acc)

        accs = lax.fori_loop(
            0, B_PER_W, body,
            tuple(jnp.zeros((L,), jnp.float32) for _ in range(N_ACC)))

        total = accs[0]
        for a in accs[1:]:
            total = total + a
        acc_v[...] = total
        pltpu.sync_copy(acc_v, out_hbm.at[wid])

    return sc_loss


_sc_loss = _make_sc_kernel()


def _tc_body(x_ref, coords_ref, centers_ref, out_ref):
    i = pl.program_id(0)
    lon = coords_ref[0, pl.ds(i * TC_BLOCK, TC_BLOCK)]
    lat = coords_ref[1, pl.ds(i * TC_BLOCK, TC_BLOCK)]
    idx = lon * GRID + lat
    onehot = jnp.where(
        jax.lax.broadcasted_iota(jnp.int32, (TC_BLOCK, N_CENTERS), 1)
        == idx[:, None],
        jnp.float32(1.0), jnp.float32(0.0))
    g = jnp.dot(onehot, centers_ref[...], preferred_element_type=jnp.float32)
    d = x_ref[...] - g
    part = jnp.sum(d * d)

    @pl.when(i == 0)
    def _():
        out_ref[0, 0] = jnp.float32(0.0)

    out_ref[0, 0] += part


_tc_loss = pl.pallas_call(
    _tc_body,
    grid=(TC_SHARE // TC_BLOCK,),
    in_specs=[
        pl.BlockSpec((TC_BLOCK, DIM), lambda i: (i, 0)),
        pl.BlockSpec((2, BATCH), lambda i: (0, 0)),
        pl.BlockSpec((N_CENTERS, DIM), lambda i: (0, 0)),
    ],
    out_specs=pl.BlockSpec((1, 1), lambda i: (0, 0), memory_space=pltpu.SMEM),
    out_shape=jax.ShapeDtypeStruct((1, 1), jnp.float32),
)


def kernel(batch_tensors, batch_coords, cluster_centers):
    coords_flat = batch_coords.reshape(-1)
    sc_partials = _sc_loss(batch_tensors, coords_flat, cluster_centers)
    tc_partial = _tc_loss(batch_tensors, batch_coords, cluster_centers)
    return (tc_partial[0, 0] + jnp.sum(sc_partials)) / jnp.float32(BATCH * DIM)


# speedup vs baseline: 1.3604x; 1.0128x over previous
import functools

import jax
import jax.numpy as jnp
from jax import lax
from jax.experimental import pallas as pl
from jax.experimental.pallas import tpu as pltpu
from jax.experimental.pallas import tpu_sc as plsc

GRID = 16
N_CENTERS = 256
DIM = 512
BATCH = 4096

TC_SHARE = 3072
SC_SHARE = BATCH - TC_SHARE
TC_BLOCK = 512

NC = 2
NS = 16
L = 16
NW = NC * NS
B_PER_W = SC_SHARE // NW
N_ACC = 8


def _make_sc_kernel():
    mesh = plsc.VectorSubcoreMesh(core_axis_name="c", subcore_axis_name="s")

    @functools.partial(
        pl.kernel,
        mesh=mesh,
        out_type=jax.ShapeDtypeStruct((NW, L), jnp.float32),
        scratch_types=[
            pltpu.VMEM((B_PER_W,), jnp.int32),
            pltpu.VMEM((B_PER_W,), jnp.int32),
            pltpu.VMEM((B_PER_W,), jnp.int32),
            pltpu.VMEM((B_PER_W, DIM), jnp.float32),
            pltpu.VMEM((B_PER_W, DIM), jnp.float32),
            pltpu.VMEM((L,), jnp.float32),
            pltpu.SemaphoreType.DMA,
            pltpu.SemaphoreType.DMA,
        ],
    )
    def sc_loss(x_hbm, coords_hbm, centers_hbm, out_hbm,
                lon_v, lat_v, idx_v, x_buf, c_buf, acc_v, sem_g, sem_x):
        wid = lax.axis_index("s") * NC + lax.axis_index("c")
        base = TC_SHARE + wid * B_PER_W
        pltpu.sync_copy(coords_hbm.at[0, pl.ds(base, B_PER_W)], lon_v)
        pltpu.sync_copy(coords_hbm.at[1, pl.ds(base, B_PER_W)], lat_v)
        for j in range(B_PER_W // L):
            s = pl.ds(j * L, L)
            idx_v[s] = lon_v[s] * GRID + lat_v[s]
        cp_g = pltpu.async_copy(centers_hbm.at[idx_v], c_buf, sem_g)
        cp_x = pltpu.async_copy(x_hbm.at[pl.ds(base, B_PER_W)], x_buf, sem_x)
        cp_g.wait()
        cp_x.wait()

        def body(r, acc):
            acc = list(acc)
            for j in range(DIM // L):
                s = pl.ds(j * L, L)
                d = x_buf[r, s] - c_buf[r, s]
                acc[j % N_ACC] = acc[j % N_ACC] + d * d
            return tuple(acc)

        accs = lax.fori_loop(
            0, B_PER_W, body,
            tuple(jnp.zeros((L,), jnp.float32) for _ in range(N_ACC)))

        total = accs[0]
        for a in accs[1:]:
            total = total + a
        acc_v[...] = total
        pltpu.sync_copy(acc_v, out_hbm.at[wid])

    return sc_loss


_sc_loss = _make_sc_kernel()


def _tc_body(x_ref, coords_ref, centers_ref, out_ref):
    i = pl.program_id(0)
    lon = coords_ref[0, pl.ds(i * TC_BLOCK, TC_BLOCK)]
    lat = coords_ref[1, pl.ds(i * TC_BLOCK, TC_BLOCK)]
    idx = lon * GRID + lat
    onehot = jnp.where(
        jax.lax.broadcasted_iota(jnp.int32, (TC_BLOCK, N_CENTERS), 1)
        == idx[:, None],
        jnp.float32(1.0), jnp.float32(0.0))
    g = jnp.dot(onehot.astype(jnp.bfloat16),
                centers_ref[...].astype(jnp.bfloat16),
                preferred_element_type=jnp.float32)
    d = x_ref[...] - g
    part = jnp.sum(d * d)

    @pl.when(i == 0)
    def _():
        out_ref[0, 0] = jnp.float32(0.0)

    out_ref[0, 0] += part


_tc_loss = pl.pallas_call(
    _tc_body,
    grid=(TC_SHARE // TC_BLOCK,),
    in_specs=[
        pl.BlockSpec((TC_BLOCK, DIM), lambda i: (i, 0)),
        pl.BlockSpec((2, BATCH), lambda i: (0, 0)),
        pl.BlockSpec((N_CENTERS, DIM), lambda i: (0, 0)),
    ],
    out_specs=pl.BlockSpec((1, 1), lambda i: (0, 0), memory_space=pltpu.SMEM),
    out_shape=jax.ShapeDtypeStruct((1, 1), jnp.float32),
)


def kernel(batch_tensors, batch_coords, cluster_centers):
    sc_partials = _sc_loss(batch_tensors, batch_coords, cluster_centers)
    tc_partial = _tc_loss(batch_tensors, batch_coords, cluster_centers)
    return (tc_partial[0, 0] + jnp.sum(sc_partials)) / jnp.float32(BATCH * DIM)
